# baseline (speedup 1.0000x reference)
import functools

import jax
import jax.numpy as jnp
from jax import lax
from jax.experimental import pallas as pl
from jax.experimental.pallas import tpu as pltpu
from jax.experimental.pallas import tpu_sc as plsc

BS, NS, C, S, H, W = 4, 2048, 512, 8, 128, 128
D = C
NQ = BS * NS
NA = NQ * S
NWORK = 32
CH = 16


def _corner_data(xv, yv, bb):
    x = jnp.clip(xv, 0.0, 1.0) * jnp.float32(W - 1)
    y = jnp.clip(yv, 0.0, 1.0) * jnp.float32(H - 1)
    xi = x.astype(jnp.int32)
    yi = y.astype(jnp.int32)
    wx = x - xi.astype(jnp.float32)
    wy = y - yi.astype(jnp.float32)
    x1 = jnp.minimum(xi + 1, W - 1)
    y1 = jnp.minimum(yi + 1, H - 1)
    r0 = bb + yi * W
    r1 = bb + y1 * W
    wxc = 1.0 - wx
    wyc = 1.0 - wy
    idx = (r0 + xi, r0 + x1, r1 + xi, r1 + x1)
    wts = (wxc, wx, wyc, wy)
    return idx, wts


def _make_sampler(n_pts, ppb):
    pw = n_pts // NWORK
    nch = pw // CH
    mesh = plsc.VectorSubcoreMesh(core_axis_name="c", subcore_axis_name="s")

    @functools.partial(
        pl.kernel,
        out_type=jax.ShapeDtypeStruct((n_pts, C), jnp.float32),
        mesh=mesh,
        scratch_types=[
            pltpu.VMEM((pw,), jnp.float32),
            pltpu.VMEM((pw,), jnp.float32),
            pltpu.VMEM((pw,), jnp.float32),
            pltpu.VMEM((4 * CH,), jnp.int32),
            pltpu.VMEM((4 * CH,), jnp.int32),
            pltpu.VMEM((4 * CH,), jnp.int32),
            pltpu.VMEM((4 * CH, C), jnp.float32),
            pltpu.VMEM((4 * CH, C), jnp.float32),
            pltpu.VMEM((4 * CH, C), jnp.float32),
            pltpu.VMEM((CH, C), jnp.float32),
            pltpu.SemaphoreType.DMA,
            pltpu.SemaphoreType.DMA,
            pltpu.SemaphoreType.DMA,
        ],
    )
    def sampler(t0, t1, t2, a0_h, a1_h, a2_h, out_h,
                a0_v, a1_v, a2_v, i0_v, i1_v, i2_v,
                r0_v, r1_v, r2_v, acc_v, sem0, sem1, sem2):
        wid = lax.axis_index("s") * 2 + lax.axis_index("c")
        base = wid * pw
        bb = (base // ppb) * (H * W)
        pltpu.sync_copy(a0_h.at[pl.ds(base, pw)], a0_v)
        pltpu.sync_copy(a1_h.at[pl.ds(base, pw)], a1_v)
        pltpu.sync_copy(a2_h.at[pl.ds(base, pw)], a2_v)

        irefs = (i0_v, i1_v, i2_v)
        rrefs = (r0_v, r1_v, r2_v)
        sems = (sem0, sem1, sem2)
        tabs = (t0, t1, t2)

        def coords_of(o):
            av0 = a0_v[pl.ds(o, CH)]
            av1 = a1_v[pl.ds(o, CH)]
            av2 = a2_v[pl.ds(o, CH)]
            return ((av0, av1), (av0, av2), (av1, av2))

        def stage_idx(g, p):
            xy = coords_of(g * CH)[p]
            idx, _ = _corner_data(xy[0], xy[1], bb)
            for cn in range(4):
                irefs[p][pl.ds(cn * CH, CH)] = idx[cn]

        def fire(p):
            pltpu.async_copy(tabs[p].at[irefs[p]], rrefs[p], sems[p])

        def drain(p):
            pltpu.make_async_copy(tabs[p].at[irefs[p]], rrefs[p], sems[p]).wait()

        for p in range(3):
            stage_idx(0, p)
            fire(p)

        def chunk_body(g, carry):
            o = g * CH
            planes = coords_of(o)
            for p in range(3):
                drain(p)
                _, wts = _corner_data(planes[p][0], planes[p][1], 0)

                def pt_body(i, carry2, p=p, wts=wts):
                    iv = jnp.zeros((16,), jnp.int32) + i
                    ws = [wts[f].at[iv].get(mode="promise_in_bounds")
                          for f in range(4)]
                    fx = (0, 1, 0, 1)
                    fy = (2, 2, 3, 3)
                    for c in range(C // 16):
                        cs = pl.ds(c * 16, 16)
                        t = None
                        for cn in range(4):
                            term = (rrefs[p][cn * CH + i, cs]
                                    * ws[fx[cn]]) * ws[fy[cn]]
                            t = term if t is None else t + term
                        if p == 0:
                            acc_v[i, cs] = t
                        else:
                            acc_v[i, cs] = acc_v[i, cs] + t
                    return carry2

                lax.fori_loop(0, CH, pt_body, 0, unroll=False)

                @pl.when(g + 1 < nch)
                def _(p=p):
                    stage_idx(g + 1, p)
                    fire(p)

            pltpu.sync_copy(acc_v, out_h.at[pl.ds(base + o, CH)])
            return carry

        lax.fori_loop(0, nch, chunk_body, 0, unroll=False)

    return sampler


def _attn_tc(feat, aux, W_q, W_k, W_v, W_out):
    blk = 256
    nq = feat.shape[0]
    grid = (nq // blk,)

    def body(f_ref, a_ref, wq_ref, wk_ref, wv_ref, wo_ref, o_ref):
        f = f_ref[...]
        a = a_ref[...]
        q = jnp.dot(f, wq_ref[...], preferred_element_type=jnp.float32)
        q = q / float(D ** (-0.5))
        k = jnp.dot(a, wk_ref[...], preferred_element_type=jnp.float32)
        v = jnp.dot(a, wv_ref[...], preferred_element_type=jnp.float32)
        qb = q.astype(jnp.bfloat16).astype(jnp.float32)
        kb = k.astype(jnp.bfloat16).astype(jnp.float32)
        k3 = kb.reshape(blk, S, D)
        sim = jnp.sum(qb[:, None, :] * k3, axis=-1)
        m = jnp.max(sim, axis=-1, keepdims=True)
        e = jnp.exp(sim - m)
        p = e / jnp.sum(e, axis=-1, keepdims=True)
        pb = p.astype(jnp.bfloat16).astype(jnp.float32)
        vb = v.astype(jnp.bfloat16).astype(jnp.float32)
        v3 = vb.reshape(blk, S, D)
        o = jnp.sum(pb[:, :, None] * v3, axis=1)
        o_ref[...] = jnp.dot(o, wo_ref[...],
                             preferred_element_type=jnp.float32) + f

    wspec = pl.BlockSpec((C, D), lambda i: (0, 0))
    return pl.pallas_call(
        body,
        grid=grid,
        in_specs=[pl.BlockSpec((blk, C), lambda i: (i, 0)),
                  pl.BlockSpec((blk * S, C), lambda i: (i, 0)),
                  wspec, wspec, wspec, wspec],
        out_specs=pl.BlockSpec((blk, D), lambda i: (i, 0)),
        out_shape=jax.ShapeDtypeStruct((nq, D), jnp.float32),
    )(feat, aux, W_q, W_k, W_v, W_out)


def _bilinear_jnp(plane, coords):
    B, Cc, Hh, Ww = plane.shape
    x = jnp.clip(coords[..., 0], 0.0, 1.0) * (Ww - 1)
    y = jnp.clip(coords[..., 1], 0.0, 1.0) * (Hh - 1)
    x0 = jnp.floor(x)
    y0 = jnp.floor(y)
    wx = (x - x0)[..., None]
    wy = (y - y0)[..., None]
    x0i = jnp.clip(x0.astype(jnp.int32), 0, Ww - 1)
    x1i = jnp.clip(x0i + 1, 0, Ww - 1)
    y0i = jnp.clip(y0.astype(jnp.int32), 0, Hh - 1)
    y1i = jnp.clip(y0i + 1, 0, Hh - 1)
    p = jnp.transpose(plane, (0, 2, 3, 1))
    gather = jax.vmap(lambda pb, yb, xb: pb[yb, xb])
    v00 = gather(p, y0i, x0i)
    v01 = gather(p, y0i, x1i)
    v10 = gather(p, y1i, x0i)
    v11 = gather(p, y1i, x1i)
    return (v00 * (1 - wx) * (1 - wy) + v01 * wx * (1 - wy)
            + v10 * (1 - wx) * wy + v11 * wx * wy)


def _sample_jnp(pos, c_xy, c_xz, c_yz):
    f_xy = _bilinear_jnp(c_xy, pos[..., jnp.array([0, 1])])
    f_xz = _bilinear_jnp(c_xz, pos[..., jnp.array([0, 2])])
    f_yz = _bilinear_jnp(c_yz, pos[..., jnp.array([1, 2])])
    return f_xy + f_xz + f_yz


def kernel(query_pos, c_xy, c_xz, c_yz, W_q, W_k, W_v, W_out, W_off, b_off):
    t_xy = jnp.transpose(c_xy, (0, 2, 3, 1)).reshape(BS * H * W, C)
    t_xz = jnp.transpose(c_xz, (0, 2, 3, 1)).reshape(BS * H * W, C)
    t_yz = jnp.transpose(c_yz, (0, 2, 3, 1)).reshape(BS * H * W, C)

    qp = query_pos.reshape(NQ, 3)
    q0, q1, q2 = qp[:, 0], qp[:, 1], qp[:, 2]

    sample_q = _make_sampler(NQ, NS)
    feat = sample_q(t_xy, t_xz, t_yz, q0, q1, q2)

    off = (feat @ W_off + b_off).reshape(BS, NS, S, 3)
    aux_pt = (off + query_pos[:, :, None, :]).reshape(NA, 3)

    sample_a = _make_sampler(NA, NS * S)
    aux = sample_a(t_xy, t_xz, t_yz, aux_pt[:, 0], aux_pt[:, 1], aux_pt[:, 2])

    out = _attn_tc(feat, aux, W_q, W_k, W_v, W_out)
    return out.reshape(BS, NS, C)
